# baseline (speedup 1.0000x reference)
import functools

import jax
import jax.numpy as jnp
from jax.experimental import pallas as pl
from jax.experimental.pallas import tpu as pltpu

_NUM_SAMPLES = 1024
_NUM_SEG = 9
_FEAT = 3072
_NUM_CLASSES = 20

_P = 32
_NCHUNK = _NUM_SAMPLES // _P
_NBUF = 4

_DN = (((1,), (1,)), ((), ()))


def _dot_t(a, w):
    return jax.lax.dot_general(a, w, _DN, preferred_element_type=jnp.float32)


def _fused_kernel(x_hbm, sf_ref, wa_ref, ba_ref, wc_ref, bc_ref, wr_ref, br_ref,
                  act_ref, comp_ref, reg_ref, xbuf, sems):
    F = _FEAT

    def copy_in(c, slot):
        return pltpu.make_async_copy(
            x_hbm.at[pl.ds(c * _P, _P)], xbuf.at[slot], sems.at[slot])

    for s in range(_NBUF):
        copy_in(s, s).start()

    def body(c, _):
        slot = jax.lax.rem(c, _NBUF)
        copy_in(c, slot).wait()
        xv = xbuf[slot]
        sf = sf_ref[pl.ds(c * _P, _P), :]
        start = (xv[:, 0, :] + xv[:, 1, :]) * (sf[:, 0:1] * 0.5)
        course = (xv[:, 2, :] + xv[:, 3, :] + xv[:, 4, :]
                  + xv[:, 5, :] + xv[:, 6, :]) * 0.2
        end = (xv[:, 7, :] + xv[:, 8, :]) * (sf[:, 1:2] * 0.5)
        nc = c + _NBUF

        @pl.when(nc < _NCHUNK)
        def _():
            copy_in(nc, slot).start()

        r = pl.ds(c * _P, _P)
        act_ref[r, :] = _dot_t(course, wa_ref[...]) + ba_ref[...]
        comp_ref[r, :] = (_dot_t(start, wc_ref[:, 0:F])
                          + _dot_t(course, wc_ref[:, F:2 * F])
                          + _dot_t(end, wc_ref[:, 2 * F:3 * F]) + bc_ref[...])
        reg_ref[r, :] = (_dot_t(start, wr_ref[:, 0:F])
                         + _dot_t(course, wr_ref[:, F:2 * F])
                         + _dot_t(end, wr_ref[:, 2 * F:3 * F]) + br_ref[...])
        return ()

    jax.lax.fori_loop(0, _NCHUNK, body, (), unroll=False)


@jax.jit
def _run(x3, sf, W_act, b_act, W_comp, b_comp, W_reg, b_reg):
    vmem = pl.BlockSpec(memory_space=pltpu.MemorySpace.VMEM)
    outs = pl.pallas_call(
        _fused_kernel,
        in_specs=[
            pl.BlockSpec(memory_space=pltpu.MemorySpace.HBM),
            vmem, vmem, vmem, vmem, vmem, vmem, vmem,
        ],
        out_specs=[vmem, vmem, vmem],
        out_shape=[
            jax.ShapeDtypeStruct((_NUM_SAMPLES, _NUM_CLASSES + 1), jnp.float32),
            jax.ShapeDtypeStruct((_NUM_SAMPLES, _NUM_CLASSES), jnp.float32),
            jax.ShapeDtypeStruct((_NUM_SAMPLES, _NUM_CLASSES * 2), jnp.float32),
        ],
        scratch_shapes=[
            pltpu.VMEM((_NBUF, _P, _NUM_SEG, _FEAT), jnp.float32),
            pltpu.SemaphoreType.DMA((_NBUF,)),
        ],
    )(x3, sf, W_act, b_act, W_comp, b_comp, W_reg, b_reg)
    return outs


def kernel(x, scale_factors, W_act, b_act, W_comp, b_comp, W_reg, b_reg):
    x3 = x.reshape(_NUM_SAMPLES, _NUM_SEG, _FEAT)
    act, comp, reg = _run(x3, scale_factors,
                          W_act, b_act.reshape(1, -1),
                          W_comp, b_comp.reshape(1, -1),
                          W_reg, b_reg.reshape(1, -1))
    return (act, comp, reg.reshape(-1, _NUM_CLASSES, 2))


# speedup vs baseline: 4.0168x; 4.0168x over previous
import functools

import jax
import jax.numpy as jnp
from jax.experimental import pallas as pl
from jax.experimental.pallas import tpu as pltpu

_NUM_SAMPLES = 1024
_NUM_SEG = 9
_FEAT = 3072
_NUM_CLASSES = 20

_DN = (((1,), (1,)), ((), ()))


def _dot_t(a, w):
    return jax.lax.dot_general(a, w, _DN, preferred_element_type=jnp.float32)


def _fused_kernel(x_ref, sf_ref, wa_ref, ba_ref, wc_ref, bc_ref, wr_ref, br_ref,
                  act_ref, comp_ref, reg_ref):
    F = _FEAT
    xb = x_ref[...]
    sf = sf_ref[...]
    xr = xb.reshape(-1, _NUM_SEG, F)
    start = (xr[:, 0, :] + xr[:, 1, :]) * (sf[:, 0:1] * 0.5)
    course = (xr[:, 2, :] + xr[:, 3, :] + xr[:, 4, :]
              + xr[:, 5, :] + xr[:, 6, :]) * 0.2
    end = (xr[:, 7, :] + xr[:, 8, :]) * (sf[:, 1:2] * 0.5)
    act_ref[...] = _dot_t(course, wa_ref[...]) + ba_ref[...]
    comp_ref[...] = (_dot_t(start, wc_ref[:, 0:F])
                     + _dot_t(course, wc_ref[:, F:2 * F])
                     + _dot_t(end, wc_ref[:, 2 * F:3 * F]) + bc_ref[...])
    reg_ref[...] = (_dot_t(start, wr_ref[:, 0:F])
                    + _dot_t(course, wr_ref[:, F:2 * F])
                    + _dot_t(end, wr_ref[:, 2 * F:3 * F]) + br_ref[...])


@functools.partial(jax.jit, static_argnames=("block",))
def _run(x, sf, W_act, b_act, W_comp, b_comp, W_reg, b_reg, block=64):
    grid = _NUM_SAMPLES // block
    nw = lambda i: (0, 0)
    outs = pl.pallas_call(
        _fused_kernel,
        grid=(grid,),
        in_specs=[
            pl.BlockSpec((block * _NUM_SEG, _FEAT), lambda i: (i, 0)),
            pl.BlockSpec((block, 2), lambda i: (i, 0)),
            pl.BlockSpec(W_act.shape, nw),
            pl.BlockSpec(b_act.shape, nw),
            pl.BlockSpec(W_comp.shape, nw),
            pl.BlockSpec(b_comp.shape, nw),
            pl.BlockSpec(W_reg.shape, nw),
            pl.BlockSpec(b_reg.shape, nw),
        ],
        out_specs=[
            pl.BlockSpec((block, _NUM_CLASSES + 1), lambda i: (i, 0)),
            pl.BlockSpec((block, _NUM_CLASSES), lambda i: (i, 0)),
            pl.BlockSpec((block, _NUM_CLASSES * 2), lambda i: (i, 0)),
        ],
        out_shape=[
            jax.ShapeDtypeStruct((_NUM_SAMPLES, _NUM_CLASSES + 1), jnp.float32),
            jax.ShapeDtypeStruct((_NUM_SAMPLES, _NUM_CLASSES), jnp.float32),
            jax.ShapeDtypeStruct((_NUM_SAMPLES, _NUM_CLASSES * 2), jnp.float32),
        ],
        compiler_params=pltpu.CompilerParams(
            dimension_semantics=("arbitrary",)),
    )(x, sf, W_act, b_act, W_comp, b_comp, W_reg, b_reg)
    return outs


def kernel(x, scale_factors, W_act, b_act, W_comp, b_comp, W_reg, b_reg):
    act, comp, reg = _run(x, scale_factors,
                          W_act, b_act.reshape(1, -1),
                          W_comp, b_comp.reshape(1, -1),
                          W_reg, b_reg.reshape(1, -1))
    return (act, comp, reg.reshape(-1, _NUM_CLASSES, 2))
